# baseline (speedup 1.0000x reference)
import functools

import jax
import jax.numpy as jnp
from jax import lax
from jax.experimental import pallas as pl
from jax.experimental.pallas import tpu as pltpu
from jax.experimental.pallas import tpu_sc as plsc

R = 128
N = 32768
K = 64
NC = 2
NS = 16
L = 16
NW = NC * NS
RPW = R // NW
GV = 32
GE = GV * L
NG = N // GE
NGV = NG // L

_BIG = 2**31 - 1


def _build(xv, m2, lane0):
    def build_group(g, carry):
        acc = xv[pl.ds(g * GE, L)]
        for j in range(1, GV):
            acc = jnp.maximum(acc, xv[pl.ds(g * GE + j * L, L)])
        gm = jnp.max(acc)
        plsc.store_scatter(
            m2, [jnp.full((L,), g, jnp.int32)],
            jnp.full((L,), gm, jnp.float32), mask=lane0)
        return carry

    lax.fori_loop(0, NG, build_group, 0)


def _step(xv, m2, outv, outi, k, lanes, lane0):
    neg = jnp.float32(-jnp.inf)
    bv = m2[pl.ds(0, L)]
    bs = lanes
    for i in range(1, NGV):
        v = m2[pl.ds(i * L, L)]
        m = v > bv
        bv = jnp.maximum(bv, v)
        bs = jnp.where(m, lanes + (i * L), bs)
    gmax = jnp.max(bv)
    gstar = jnp.min(jnp.where(bv == gmax, bs, _BIG))
    base = gstar * GE
    iacc = jnp.full((L,), _BIG, jnp.int32)
    for j in range(GV):
        v = xv[pl.ds(base + j * L, L)]
        iacc = jnp.minimum(
            iacc, jnp.where(v == gmax, lanes + (base + j * L), _BIG))
    bi = jnp.min(iacc)
    kidx = jnp.full((L,), k, jnp.int32)
    plsc.store_scatter(outv, [kidx], jnp.full((L,), gmax, jnp.float32),
                       mask=lane0)
    plsc.store_scatter(outi, [kidx], jnp.full((L,), bi, jnp.int32),
                       mask=lane0)
    q = (bi >> 4) << 4
    lane = bi - q
    vq = xv[pl.ds(q, L)]
    xv[pl.ds(q, L)] = jnp.where(lanes == lane, neg, vq)
    acc2 = xv[pl.ds(base, L)]
    for j in range(1, GV):
        acc2 = jnp.maximum(acc2, xv[pl.ds(base + j * L, L)])
    plsc.store_scatter(
        m2, [jnp.full((L,), gstar, jnp.int32)],
        jnp.full((L,), jnp.max(acc2), jnp.float32), mask=lane0)


def _extract_pair(xva, m2a, outva, outia, xvb, m2b, outvb, outib,
                  lanes, lane0):
    def extract(k, carry):
        _step(xva, m2a, outva, outia, k, lanes, lane0)
        _step(xvb, m2b, outvb, outib, k, lanes, lane0)
        return carry

    lax.fori_loop(0, K, extract, 0)


@functools.partial(
    pl.kernel,
    out_type=(
        jax.ShapeDtypeStruct((R, K), jnp.float32),
        jax.ShapeDtypeStruct((R, K), jnp.int32),
    ),
    mesh=plsc.VectorSubcoreMesh(
        core_axis_name="c", subcore_axis_name="s",
        num_cores=NC, num_subcores=NS),
    compiler_params=pltpu.CompilerParams(needs_layout_passes=False),
    scratch_types=[
        pltpu.VMEM((N,), jnp.float32),
        pltpu.VMEM((N,), jnp.float32),
        pltpu.VMEM((N,), jnp.float32),
        pltpu.VMEM((NG,), jnp.float32),
        pltpu.VMEM((NG,), jnp.float32),
        pltpu.VMEM((K,), jnp.float32),
        pltpu.VMEM((K,), jnp.int32),
        pltpu.VMEM((K,), jnp.float32),
        pltpu.VMEM((K,), jnp.int32),
        pltpu.SemaphoreType.DMA,
        pltpu.SemaphoreType.DMA,
        pltpu.SemaphoreType.DMA,
    ],
)
def _topk_sc(x_hbm, vals_hbm, idx_hbm, bufa, bufb, bufc, m2a, m2b,
             outva, outia, outvb, outib, sema, semb, semc):
    wid = lax.axis_index("s") * NC + lax.axis_index("c")
    lanes = lax.iota(jnp.int32, L)
    lane0 = lanes == 0
    r0 = wid * RPW

    def flush(row_lo):
        pltpu.sync_copy(outva, vals_hbm.at[row_lo])
        pltpu.sync_copy(outia, idx_hbm.at[row_lo])
        pltpu.sync_copy(outvb, vals_hbm.at[row_lo + 1])
        pltpu.sync_copy(outib, idx_hbm.at[row_lo + 1])

    h0 = pltpu.async_copy(x_hbm.at[r0], bufa, sema)
    h1 = pltpu.async_copy(x_hbm.at[r0 + 1], bufb, semb)
    h0.wait()
    _build(bufa, m2a, lane0)
    h2 = pltpu.async_copy(x_hbm.at[r0 + 2], bufc, semc)
    h1.wait()
    _build(bufb, m2b, lane0)
    _extract_pair(bufa, m2a, outva, outia, bufb, m2b, outvb, outib,
                  lanes, lane0)
    flush(r0)
    h3 = pltpu.async_copy(x_hbm.at[r0 + 3], bufa, sema)
    h2.wait()
    _build(bufc, m2a, lane0)
    h3.wait()
    _build(bufa, m2b, lane0)
    _extract_pair(bufc, m2a, outva, outia, bufa, m2b, outvb, outib,
                  lanes, lane0)
    flush(r0 + 2)


def kernel(x):
    vals, idx = _topk_sc(x)
    return vals, idx.astype(jnp.int64)


# speedup vs baseline: 1.2157x; 1.2157x over previous
import functools

import jax
import jax.numpy as jnp
from jax import lax
from jax.experimental import pallas as pl
from jax.experimental.pallas import tpu as pltpu
from jax.experimental.pallas import tpu_sc as plsc

R = 128
N = 32768
K = 64
NC = 2
NS = 16
L = 16
NW = NC * NS
RPW = R // NW
GV = 16
GE = GV * L
NG = N // GE
NGV = NG // L

_BIG = 2**31 - 1


def _build(xv, m2, lane0):
    def build_group(g, carry):
        acc = xv[pl.ds(g * GE, L)]
        for j in range(1, GV):
            acc = jnp.maximum(acc, xv[pl.ds(g * GE + j * L, L)])
        gm = jnp.max(acc)
        plsc.store_scatter(
            m2, [jnp.full((L,), g, jnp.int32)],
            jnp.full((L,), gm, jnp.float32), mask=lane0)
        return carry

    lax.fori_loop(0, NG, build_group, 0)


def _step(xv, m2, outv, outi, k, lanes, lane0):
    neg = jnp.float32(-jnp.inf)
    bv = m2[pl.ds(0, L)]
    bs = lanes
    for i in range(1, NGV):
        v = m2[pl.ds(i * L, L)]
        m = v > bv
        bv = jnp.maximum(bv, v)
        bs = jnp.where(m, lanes + (i * L), bs)
    gmax = jnp.max(bv)
    gstar = jnp.min(jnp.where(bv == gmax, bs, _BIG))
    base = gstar * GE
    iacc = jnp.full((L,), _BIG, jnp.int32)
    macc = jnp.full((L,), neg, jnp.float32)
    ccc = jnp.zeros((L,), jnp.int32)
    for j in range(GV):
        v = xv[pl.ds(base + j * L, L)]
        eq = v == gmax
        iacc = jnp.minimum(iacc, jnp.where(eq, lanes + (base + j * L), _BIG))
        macc = jnp.maximum(macc, jnp.where(eq, neg, v))
        ccc = ccc + jnp.where(eq, 1, 0)
    bi = jnp.min(iacc)
    newmax = jnp.where(jnp.sum(ccc) > 1, gmax, jnp.max(macc))
    kidx = jnp.full((L,), k, jnp.int32)
    plsc.store_scatter(outv, [kidx], jnp.full((L,), gmax, jnp.float32),
                       mask=lane0)
    plsc.store_scatter(outi, [kidx], jnp.full((L,), bi, jnp.int32),
                       mask=lane0)
    q = (bi >> 4) << 4
    lane = bi - q
    vq = xv[pl.ds(q, L)]
    xv[pl.ds(q, L)] = jnp.where(lanes == lane, neg, vq)
    plsc.store_scatter(
        m2, [jnp.full((L,), gstar, jnp.int32)],
        jnp.full((L,), newmax, jnp.float32), mask=lane0)


def _extract_pair(xva, m2a, outva, outia, xvb, m2b, outvb, outib,
                  lanes, lane0):
    def extract(k, carry):
        _step(xva, m2a, outva, outia, k, lanes, lane0)
        _step(xvb, m2b, outvb, outib, k, lanes, lane0)
        return carry

    lax.fori_loop(0, K, extract, 0)


@functools.partial(
    pl.kernel,
    out_type=(
        jax.ShapeDtypeStruct((R, K), jnp.float32),
        jax.ShapeDtypeStruct((R, K), jnp.int32),
    ),
    mesh=plsc.VectorSubcoreMesh(
        core_axis_name="c", subcore_axis_name="s",
        num_cores=NC, num_subcores=NS),
    compiler_params=pltpu.CompilerParams(needs_layout_passes=False),
    scratch_types=[
        pltpu.VMEM((N,), jnp.float32),
        pltpu.VMEM((N,), jnp.float32),
        pltpu.VMEM((N,), jnp.float32),
        pltpu.VMEM((NG,), jnp.float32),
        pltpu.VMEM((NG,), jnp.float32),
        pltpu.VMEM((K,), jnp.float32),
        pltpu.VMEM((K,), jnp.int32),
        pltpu.VMEM((K,), jnp.float32),
        pltpu.VMEM((K,), jnp.int32),
        pltpu.SemaphoreType.DMA,
        pltpu.SemaphoreType.DMA,
        pltpu.SemaphoreType.DMA,
    ],
)
def _topk_sc(x_hbm, vals_hbm, idx_hbm, bufa, bufb, bufc, m2a, m2b,
             outva, outia, outvb, outib, sema, semb, semc):
    wid = lax.axis_index("s") * NC + lax.axis_index("c")
    lanes = lax.iota(jnp.int32, L)
    lane0 = lanes == 0
    r0 = wid * RPW

    def flush(row_lo):
        pltpu.sync_copy(outva, vals_hbm.at[row_lo])
        pltpu.sync_copy(outia, idx_hbm.at[row_lo])
        pltpu.sync_copy(outvb, vals_hbm.at[row_lo + 1])
        pltpu.sync_copy(outib, idx_hbm.at[row_lo + 1])

    h0 = pltpu.async_copy(x_hbm.at[r0], bufa, sema)
    h1 = pltpu.async_copy(x_hbm.at[r0 + 1], bufb, semb)
    h0.wait()
    _build(bufa, m2a, lane0)
    h2 = pltpu.async_copy(x_hbm.at[r0 + 2], bufc, semc)
    h1.wait()
    _build(bufb, m2b, lane0)
    _extract_pair(bufa, m2a, outva, outia, bufb, m2b, outvb, outib,
                  lanes, lane0)
    flush(r0)
    h3 = pltpu.async_copy(x_hbm.at[r0 + 3], bufa, sema)
    h2.wait()
    _build(bufc, m2a, lane0)
    h3.wait()
    _build(bufa, m2b, lane0)
    _extract_pair(bufc, m2a, outva, outia, bufa, m2b, outvb, outib,
                  lanes, lane0)
    flush(r0 + 2)


def kernel(x):
    vals, idx = _topk_sc(x)
    return vals, idx.astype(jnp.int64)
